# Initial kernel scaffold; baseline (speedup 1.0000x reference)
import jax
import jax.numpy as jnp
from jax.experimental import pallas as pl


def kernel(input, target):
    raise NotImplementedError("write your pallas kernel here")


# speedup vs baseline: 5.5742x; 5.5742x over previous
import functools

import numpy as np
import jax
import jax.numpy as jnp
from jax import lax
from jax.experimental import pallas as pl
from jax.experimental.pallas import tpu as pltpu
from jax.experimental.pallas import tpu_sc as plsc

B = 64
C = 80
CPAD = 128
D = 128
P = B * (B - 1) // 2
PPAD = 2048

_i0, _i1 = np.triu_indices(B, k=1)
_FLAT_IDX = np.zeros((PPAD,), np.int32)
_FLAT_IDX[:P] = (_i0 * B + _i1).astype(np.int32)

_KC = 16


def _gram_body(x_ref, out_ref):
    x = x_ref[...]
    g = lax.dot_general(
        x, x, (((2,), (2,)), ((0,), (0,))),
        preferred_element_type=jnp.float32)
    nsq = jnp.sum(x * x, axis=2)
    den = nsq[:, :, None] * nsq[:, None, :]
    out_ref[...] = g * lax.rsqrt(jnp.maximum(den, 1e-18))


def _gram_tc(xt):
    return pl.pallas_call(
        _gram_body,
        grid=(C // _KC,),
        in_specs=[pl.BlockSpec((_KC, B, D), lambda i: (i, 0, 0))],
        out_specs=pl.BlockSpec((_KC, B, B), lambda i: (i, 0, 0)),
        out_shape=jax.ShapeDtypeStruct((C, B, B), jnp.float32),
    )(xt)


_NC = 2
_NS = 16
_NW = _NC * _NS
_BPW = PPAD // _NW

_mesh = plsc.VectorSubcoreMesh(core_axis_name="c", subcore_axis_name="s")


@functools.partial(
    pl.kernel,
    mesh=_mesh,
    out_type=jax.ShapeDtypeStruct((PPAD, CPAD), jnp.float32),
    scratch_types=[
        pltpu.VMEM((_BPW,), jnp.int32),
        pltpu.VMEM((_BPW, CPAD), jnp.float32),
        pltpu.SemaphoreType.DMA,
    ],
)
def _pair_gather_sc(table_hbm, idx_hbm, out_hbm, idx_v, rows_v, sem):
    wid = lax.axis_index("s") * _NC + lax.axis_index("c")
    base = wid * _BPW
    pltpu.sync_copy(idx_hbm.at[pl.ds(base, _BPW)], idx_v)
    pltpu.async_copy(table_hbm.at[idx_v], rows_v, sem).wait()
    pltpu.sync_copy(rows_v, out_hbm.at[pl.ds(base, _BPW)])


def kernel(input, target):
    xt = jnp.transpose(input, (1, 0, 2))
    gn = _gram_tc(xt)
    table = jnp.pad(gn.reshape(C, B * B),
                    ((0, CPAD - C), (0, 0))).T
    idx = jnp.asarray(_FLAT_IDX)
    out = _pair_gather_sc(table, idx)
    return out[:P, :C]


# speedup vs baseline: 6.4054x; 1.1491x over previous
import functools

import numpy as np
import jax
import jax.numpy as jnp
from jax import lax
from jax.experimental import pallas as pl
from jax.experimental.pallas import tpu as pltpu
from jax.experimental.pallas import tpu_sc as plsc

B = 64
C = 80
CPAD = 128
D = 128
P = B * (B - 1) // 2
PPAD = 2048

_i0, _i1 = np.triu_indices(B, k=1)
_FLAT_IDX = np.zeros((PPAD,), np.int32)
_FLAT_IDX[:P] = (_i0 * B + _i1).astype(np.int32)


def _gram_body(x_ref, out_ref):
    x = x_ref[...]
    g = lax.dot_general(
        x, x, (((2,), (2,)), ((1,), (1,))),
        preferred_element_type=jnp.float32)
    nsq = jnp.sum(x * x, axis=2).T
    den = nsq[:, :, None] * nsq[:, None, :]
    gn = g * lax.rsqrt(jnp.maximum(den, 1e-18))
    t = gn.reshape(C, B * B).T
    out_ref[:, :C] = t


def _gram_tc(x):
    return pl.pallas_call(
        _gram_body,
        out_shape=jax.ShapeDtypeStruct((B * B, CPAD), jnp.float32),
    )(x)


_NC = 2
_NS = 16
_NW = _NC * _NS
_BPW = PPAD // _NW

_mesh = plsc.VectorSubcoreMesh(core_axis_name="c", subcore_axis_name="s")


@functools.partial(
    pl.kernel,
    mesh=_mesh,
    out_type=jax.ShapeDtypeStruct((PPAD, CPAD), jnp.float32),
    scratch_types=[
        pltpu.VMEM((_BPW,), jnp.int32),
        pltpu.VMEM((_BPW, CPAD), jnp.float32),
        pltpu.SemaphoreType.DMA,
    ],
)
def _pair_gather_sc(table_hbm, idx_hbm, out_hbm, idx_v, rows_v, sem):
    wid = lax.axis_index("s") * _NC + lax.axis_index("c")
    base = wid * _BPW
    pltpu.sync_copy(idx_hbm.at[pl.ds(base, _BPW)], idx_v)
    pltpu.async_copy(table_hbm.at[idx_v], rows_v, sem).wait()
    pltpu.sync_copy(rows_v, out_hbm.at[pl.ds(base, _BPW)])


def kernel(input, target):
    table = _gram_tc(input)
    idx = jnp.asarray(_FLAT_IDX)
    out = _pair_gather_sc(table, idx)
    return out[:P, :C]
